# baseline (speedup 1.0000x reference)
import functools

import jax
import jax.numpy as jnp
from jax import lax
from jax.experimental import pallas as pl
from jax.experimental.pallas import tpu as pltpu
from jax.experimental.pallas import tpu_sc as plsc

_DIM = 32
_NE = 8192
_COMMIT = 0.25
_BLK = 1024
_CB = 2048

_NC = 2
_NS = 16
_NW = _NC * _NS
_ROWS = 8192
_BPW = _ROWS // _NW


def _dist_argmin(z2_ref, f_ref, e_ref, e2_ref, idx_ref):
    fb = f_ref[...]
    Eb = e_ref[...]
    z2 = z2_ref[0, 0, :].reshape(_BLK, 1)
    e2 = e2_ref[...]

    a = (2.0 * fb).astype(jnp.bfloat16)
    E16 = Eb.astype(jnp.bfloat16)
    m = jnp.dot(a, E16, preferred_element_type=jnp.float32)
    dist = z2 - m + e2

    lane_f = lax.broadcasted_iota(jnp.int32, (_BLK, 128), 1).astype(jnp.float32)
    accf = None
    acci = None
    ng = _CB // 128
    for b in range(_NE // _CB):
        colmin = dist[:, b * _CB:b * _CB + 128]
        colgid = jnp.zeros((_BLK, 128), jnp.float32)
        for g in range(1, ng):
            grp = dist[:, b * _CB + g * 128:b * _CB + (g + 1) * 128]
            take = grp < colmin
            colmin = jnp.where(take, grp, colmin)
            colgid = jnp.where(take, float(g), colgid)
        tmin = jnp.min(colmin, axis=1)
        cand = jnp.where(colmin == tmin[:, None],
                         colgid * 128.0 + lane_f, float(_NE))
        tidx = jnp.min(cand, axis=1).astype(jnp.int32) + b * _CB
        tround = tmin.astype(jnp.bfloat16).astype(jnp.float32)
        if b == 0:
            accf, acci = tround, tidx
        else:
            take = tmin < accf
            accf = jnp.where(take, tround, accf)
            acci = jnp.where(take, tidx, acci)
    idx_ref[...] = acci.reshape(1, 1, _BLK)


def _sc_gather_body(table_hbm, idx_hbm, out_hbm, idx_v, rows_v, sem):
    wid = lax.axis_index("s") * _NC + lax.axis_index("c")
    base = wid * _BPW
    pltpu.sync_copy(idx_hbm.at[pl.ds(base, _BPW)], idx_v)
    pltpu.async_copy(table_hbm.at[idx_v], rows_v, sem).wait()
    pltpu.sync_copy(rows_v, out_hbm.at[pl.ds(base, _BPW)])


_QROW = 128
_sc_gather = pl.kernel(
    _sc_gather_body,
    out_type=jax.ShapeDtypeStruct((_ROWS, _QROW), jnp.float32),
    mesh=plsc.VectorSubcoreMesh(core_axis_name="c", subcore_axis_name="s"),
    scratch_types=[
        pltpu.VMEM((_BPW,), jnp.int32),
        pltpu.VMEM((_BPW, _QROW), jnp.float32),
        pltpu.SemaphoreType.DMA,
    ],
)

_EBLK = 4096


def _epilogue(f_ref, q4_ref, idx_ref, qout_ref, dsum_ref):
    i = pl.program_id(0)
    fb = f_ref[...]
    q4 = q4_ref[...]
    sel = (idx_ref[0, 0, :] & 3).reshape(_EBLK, 1)
    qb = jnp.where(
        sel == 0, q4[:, 0:32],
        jnp.where(sel == 1, q4[:, 32:64],
                  jnp.where(sel == 2, q4[:, 64:96], q4[:, 96:128])))
    qd = qb - fb
    qout_ref[...] = fb + qd
    part = jnp.sum(qd * qd).reshape(1, 1)

    @pl.when(i == 0)
    def _():
        dsum_ref[...] = jnp.zeros_like(dsum_ref)

    dsum_ref[...] += part


def kernel(z, embed):
    rows = z.shape[0] * z.shape[1]
    f = z.reshape(rows, _DIM)
    z2 = (z ** 2).sum(axis=2).reshape(rows // _BLK, 1, _BLK)
    e2 = (embed ** 2).sum(axis=0, keepdims=True)
    nblk = rows // _BLK
    idx = pl.pallas_call(
        _dist_argmin,
        grid=(nblk,),
        in_specs=[
            pl.BlockSpec((1, 1, _BLK), lambda i: (i, 0, 0)),
            pl.BlockSpec((_BLK, _DIM), lambda i: (i, 0)),
            pl.BlockSpec((_DIM, _NE), lambda i: (0, 0)),
            pl.BlockSpec((1, _NE), lambda i: (0, 0)),
        ],
        out_specs=pl.BlockSpec((1, 1, _BLK), lambda i: (i, 0, 0)),
        out_shape=jax.ShapeDtypeStruct((nblk, 1, _BLK), jnp.int32),
    )(z2, f, embed, e2)
    idx_flat = idx.reshape(rows)
    table4 = embed.T.reshape(_NE // 4, _QROW)
    q4 = _sc_gather(table4, idx_flat >> 2)
    idx3 = idx_flat.reshape(rows // _EBLK, 1, _EBLK)
    qout, dsum = pl.pallas_call(
        _epilogue,
        grid=(rows // _EBLK,),
        in_specs=[
            pl.BlockSpec((_EBLK, _DIM), lambda i: (i, 0)),
            pl.BlockSpec((_EBLK, _QROW), lambda i: (i, 0)),
            pl.BlockSpec((1, 1, _EBLK), lambda i: (i, 0, 0)),
        ],
        out_specs=[
            pl.BlockSpec((_EBLK, _DIM), lambda i: (i, 0)),
            pl.BlockSpec((1, 1), lambda i: (0, 0)),
        ],
        out_shape=[
            jax.ShapeDtypeStruct((rows, _DIM), jnp.float32),
            jax.ShapeDtypeStruct((1, 1), jnp.float32),
        ],
    )(f, q4, idx3)
    quantize = qout.reshape(z.shape)
    diff = (_COMMIT / (rows * _DIM)) * dsum[0, 0]
    embed_ind = idx_flat.reshape(z.shape[:-1])
    return (quantize, diff, embed_ind, embed)


# speedup vs baseline: 1.0106x; 1.0106x over previous
import functools

import jax
import jax.numpy as jnp
from jax import lax
from jax.experimental import pallas as pl
from jax.experimental.pallas import tpu as pltpu
from jax.experimental.pallas import tpu_sc as plsc

_DIM = 32
_NE = 8192
_COMMIT = 0.25
_BLK = 1024
_CB = 2048

_NC = 2
_NS = 16
_NW = _NC * _NS
_ROWS = 8192
_BPW = _ROWS // _NW


def _dist_argmin(z2_ref, f_ref, e_ref, e2_ref, idx_ref):
    fb = f_ref[...]
    Eb = e_ref[...]
    z2 = z2_ref[0, 0, :].reshape(_BLK, 1)
    e2 = e2_ref[...]

    a = (2.0 * fb).astype(jnp.bfloat16)
    E16 = Eb.astype(jnp.bfloat16)
    m = jnp.dot(a, E16, preferred_element_type=jnp.float32)

    lane_f = lax.broadcasted_iota(jnp.int32, (_BLK, 128), 1).astype(jnp.float32)
    accf = None
    acci = None
    ng = _CB // 128
    for b in range(_NE // _CB):
        c0 = b * _CB
        colmin = z2 - m[:, c0:c0 + 128] + e2[:, c0:c0 + 128]
        colgid = jnp.zeros((_BLK, 128), jnp.float32)
        for g in range(1, ng):
            s = c0 + g * 128
            grp = z2 - m[:, s:s + 128] + e2[:, s:s + 128]
            take = grp < colmin
            colmin = jnp.where(take, grp, colmin)
            colgid = jnp.where(take, float(g), colgid)
        tmin = jnp.min(colmin, axis=1)
        cand = jnp.where(colmin == tmin[:, None],
                         colgid * 128.0 + lane_f, float(_NE))
        tidx = jnp.min(cand, axis=1).astype(jnp.int32) + b * _CB
        tround = tmin.astype(jnp.bfloat16).astype(jnp.float32)
        if b == 0:
            accf, acci = tround, tidx
        else:
            take = tmin < accf
            accf = jnp.where(take, tround, accf)
            acci = jnp.where(take, tidx, acci)
    idx_ref[...] = acci.reshape(1, 1, _BLK)


def _sc_gather_body(table_hbm, idx_hbm, out_hbm, idx_v, rows_v, sem):
    wid = lax.axis_index("s") * _NC + lax.axis_index("c")
    base = wid * _BPW
    pltpu.sync_copy(idx_hbm.at[pl.ds(base, _BPW)], idx_v)
    pltpu.async_copy(table_hbm.at[idx_v], rows_v, sem).wait()
    pltpu.sync_copy(rows_v, out_hbm.at[pl.ds(base, _BPW)])


_QROW = 128
_sc_gather = pl.kernel(
    _sc_gather_body,
    out_type=jax.ShapeDtypeStruct((_ROWS, _QROW), jnp.float32),
    mesh=plsc.VectorSubcoreMesh(core_axis_name="c", subcore_axis_name="s"),
    scratch_types=[
        pltpu.VMEM((_BPW,), jnp.int32),
        pltpu.VMEM((_BPW, _QROW), jnp.float32),
        pltpu.SemaphoreType.DMA,
    ],
)

_EBLK = 4096


def _epilogue(f_ref, q4_ref, idx_ref, qout_ref, dsum_ref):
    i = pl.program_id(0)
    fb = f_ref[...]
    q4 = q4_ref[...]
    sel = (idx_ref[0, 0, :] & 3).reshape(_EBLK, 1)
    qb = jnp.where(
        sel == 0, q4[:, 0:32],
        jnp.where(sel == 1, q4[:, 32:64],
                  jnp.where(sel == 2, q4[:, 64:96], q4[:, 96:128])))
    qd = qb - fb
    qout_ref[...] = fb + qd
    part = jnp.sum(qd * qd).reshape(1, 1)

    @pl.when(i == 0)
    def _():
        dsum_ref[...] = jnp.zeros_like(dsum_ref)

    dsum_ref[...] += part


def kernel(z, embed):
    rows = z.shape[0] * z.shape[1]
    f = z.reshape(rows, _DIM)
    z2 = (z ** 2).sum(axis=2).reshape(rows // _BLK, 1, _BLK)
    e2 = (embed ** 2).sum(axis=0, keepdims=True)
    nblk = rows // _BLK
    idx = pl.pallas_call(
        _dist_argmin,
        grid=(nblk,),
        in_specs=[
            pl.BlockSpec((1, 1, _BLK), lambda i: (i, 0, 0)),
            pl.BlockSpec((_BLK, _DIM), lambda i: (i, 0)),
            pl.BlockSpec((_DIM, _NE), lambda i: (0, 0)),
            pl.BlockSpec((1, _NE), lambda i: (0, 0)),
        ],
        out_specs=pl.BlockSpec((1, 1, _BLK), lambda i: (i, 0, 0)),
        out_shape=jax.ShapeDtypeStruct((nblk, 1, _BLK), jnp.int32),
    )(z2, f, embed, e2)
    idx_flat = idx.reshape(rows)
    table4 = embed.T.reshape(_NE // 4, _QROW)
    q4 = _sc_gather(table4, idx_flat >> 2)
    idx3 = idx_flat.reshape(rows // _EBLK, 1, _EBLK)
    qout, dsum = pl.pallas_call(
        _epilogue,
        grid=(rows // _EBLK,),
        in_specs=[
            pl.BlockSpec((_EBLK, _DIM), lambda i: (i, 0)),
            pl.BlockSpec((_EBLK, _QROW), lambda i: (i, 0)),
            pl.BlockSpec((1, 1, _EBLK), lambda i: (i, 0, 0)),
        ],
        out_specs=[
            pl.BlockSpec((_EBLK, _DIM), lambda i: (i, 0)),
            pl.BlockSpec((1, 1), lambda i: (0, 0)),
        ],
        out_shape=[
            jax.ShapeDtypeStruct((rows, _DIM), jnp.float32),
            jax.ShapeDtypeStruct((1, 1), jnp.float32),
        ],
    )(f, q4, idx3)
    quantize = qout.reshape(z.shape)
    diff = (_COMMIT / (rows * _DIM)) * dsum[0, 0]
    embed_ind = idx_flat.reshape(z.shape[:-1])
    return (quantize, diff, embed_ind, embed)
